# baseline (speedup 1.0000x reference)
import functools

import jax
import jax.numpy as jnp
from jax import lax
from jax.experimental import pallas as pl
from jax.experimental.pallas import tpu as pltpu
from jax.experimental.pallas import tpu_sc as plsc

NUM_TOKENS = 32768
NSRC = 32768
IN_FEAT = 128
DIM = 256
NC, NS, L = 2, 16, 16
NW = NC * NS
TOK_PER_W = NUM_TOKENS // NW
CHUNK = 64
NCHUNK = TOK_PER_W // CHUNK
IDX_BITS = 15


def _mm_body(x_ref, w_ref, b_ref, o_ref):
    o_ref[...] = (
        jnp.dot(x_ref[...], w_ref[...], preferred_element_type=jnp.float32)
        + b_ref[...]
    )


def _matmul(x, W, b):
    M = x.shape[0]
    BM = 1024
    return pl.pallas_call(
        _mm_body,
        grid=(M // BM,),
        in_specs=[
            pl.BlockSpec((BM, IN_FEAT), lambda i: (i, 0)),
            pl.BlockSpec((IN_FEAT, DIM), lambda i: (0, 0)),
            pl.BlockSpec((1, DIM), lambda i: (0, 0)),
        ],
        out_specs=pl.BlockSpec((BM, DIM), lambda i: (i, 0)),
        out_shape=jax.ShapeDtypeStruct((M, DIM), jnp.float32),
    )(x, W, b.reshape(1, DIM))


_mesh = plsc.VectorSubcoreMesh(core_axis_name="c", subcore_axis_name="s")


@functools.partial(
    pl.kernel,
    out_type=jax.ShapeDtypeStruct((NUM_TOKENS, DIM), jnp.float32),
    mesh=_mesh,
    compiler_params=pltpu.CompilerParams(
        needs_layout_passes=False, use_tc_tiling_on_sc=False),
    scratch_types=[
        pltpu.VMEM((NSRC,), jnp.int32),
        pltpu.VMEM((TOK_PER_W,), jnp.int32),
        pltpu.VMEM((NCHUNK, CHUNK), jnp.int32),
        pltpu.VMEM((NCHUNK, CHUNK), jnp.int32),
        pltpu.VMEM((TOK_PER_W + L,), jnp.float32),
        pltpu.VMEM((2, CHUNK, DIM), jnp.float32),
        pltpu.VMEM((2, CHUNK, DIM), jnp.float32),
        pltpu.SemaphoreType.DMA,
        pltpu.SemaphoreType.DMA,
    ],
)
def _sc_scatter(sidx_hbm, peidx_hbm, x_hbm, pe_hbm, out_hbm,
                idx_v, winner_v, wc_v, pw_v, valid_v, xrows_v, perows_v,
                semx, semp):
    wid = lax.axis_index("s") * NC + lax.axis_index("c")
    base = wid * TOK_PER_W

    pltpu.sync_copy(sidx_hbm, idx_v)
    neg1 = jnp.full((L,), -1, jnp.int32)

    def init_body(g, _):
        winner_v[pl.ds(g * L, L)] = neg1
        return 0

    lax.fori_loop(0, TOK_PER_W // L, init_body, 0)

    sent = jnp.int32(2**31 - 1)
    iota = lax.iota(jnp.int32, L)
    shift_idx = jnp.minimum(iota + 1, L - 1)
    last_lane = iota == (L - 1)

    def scan_body(g, _):
        idx16 = idx_v[pl.ds(g * L, L)]
        local = idx16 - base
        inr = (local >= 0) & (local < TOK_PER_W)
        i_vec = g * L + iota
        key = jnp.where(inr, (local << IDX_BITS) | i_vec, sent)
        skey, _ = plsc.sort_key_val(key, key)
        nxt = skey.at[shift_idx].get(mode="promise_in_bounds")
        tok = skey >> IDX_BITS
        keep = ((tok != (nxt >> IDX_BITS)) | last_lane) & (skey != sent)
        tok_st = tok & (TOK_PER_W - 1)
        ival = skey & (NSRC - 1)
        plsc.store_scatter(winner_v, [tok_st], ival, mask=keep)
        return 0

    lax.fori_loop(0, NSRC // L, scan_body, 0)

    pltpu.sync_copy(peidx_hbm, idx_v)

    def pw_body(g, nc_acc):
        sl = pl.ds(g * L, L)
        w16 = winner_v[sl]
        valid = w16 >= 0
        wcl = jnp.maximum(w16, 0)
        c = g // (CHUNK // L)
        o = (g % (CHUNK // L)) * L
        csl = pl.ds(o, L)
        wc_v[c, csl] = wcl
        pw = plsc.load_gather(idx_v, [wcl])
        pw_v[c, csl] = pw
        valid_v[sl] = jnp.where(valid, 1.0, 0.0).astype(jnp.float32)
        return nc_acc

    lax.fori_loop(0, TOK_PER_W // L, pw_body, 0)

    def issue(c):
        p = c % 2
        gx = pltpu.async_copy(x_hbm.at[wc_v.at[c]], xrows_v.at[p], semx)
        gp = pltpu.async_copy(pe_hbm.at[pw_v.at[c]], perows_v.at[p], semp)
        return gx, gp

    pend = issue(0)
    for c in range(NCHUNK):
        p = c % 2
        pend[0].wait()
        pend[1].wait()
        if c + 1 < NCHUNK:
            pend = issue(c + 1)

        def row_body(r, _, c=c, p=p):
            fvec = valid_v[pl.ds(c * CHUNK + r, L)]
            fv = jnp.broadcast_to(fvec[0], (L,))
            for j in range(DIM // L):
                sl = pl.ds(j * L, L)
                xrows_v[p, r, sl] = (
                    xrows_v[p, r, sl] + perows_v[p, r, sl]) * fv
            return 0

        lax.fori_loop(0, CHUNK, row_body, 0)
        pltpu.sync_copy(
            xrows_v.at[p], out_hbm.at[pl.ds(base + c * CHUNK, CHUNK)])


def kernel(sdata, scatter_idxs, pe_idxs, pe_embed, W, b):
    x = _matmul(sdata.reshape(-1, IN_FEAT), W, b)
    return _sc_scatter(
        scatter_idxs.astype(jnp.int32), pe_idxs.astype(jnp.int32), x, pe_embed)


# speedup vs baseline: 1.1472x; 1.1472x over previous
import functools

import jax
import jax.numpy as jnp
from jax import lax
from jax.experimental import pallas as pl
from jax.experimental.pallas import tpu as pltpu
from jax.experimental.pallas import tpu_sc as plsc

NUM_TOKENS = 32768
NSRC = 32768
IN_FEAT = 128
DIM = 256
NC, NS, L = 2, 16, 16
NW = NC * NS
TOK_PER_W = NUM_TOKENS // NW
CHUNK = 64
NCHUNK = TOK_PER_W // CHUNK
IDX_BITS = 15


def _mm_body(x_ref, w_ref, b_ref, o_ref):
    o_ref[...] = (
        jnp.dot(x_ref[...], w_ref[...], preferred_element_type=jnp.float32)
        + b_ref[...]
    )


def _matmul(x, W, b):
    M = x.shape[0]
    BM = 1024
    return pl.pallas_call(
        _mm_body,
        grid=(M // BM,),
        in_specs=[
            pl.BlockSpec((BM, IN_FEAT), lambda i: (i, 0)),
            pl.BlockSpec((IN_FEAT, DIM), lambda i: (0, 0)),
            pl.BlockSpec((1, DIM), lambda i: (0, 0)),
        ],
        out_specs=pl.BlockSpec((BM, DIM), lambda i: (i, 0)),
        out_shape=jax.ShapeDtypeStruct((M, DIM), jnp.float32),
    )(x, W, b.reshape(1, DIM))


_mesh = plsc.VectorSubcoreMesh(core_axis_name="c", subcore_axis_name="s")


@functools.partial(
    pl.kernel,
    out_type=jax.ShapeDtypeStruct((NUM_TOKENS, DIM), jnp.float32),
    mesh=_mesh,
    compiler_params=pltpu.CompilerParams(needs_layout_passes=False),
    scratch_types=[
        pltpu.VMEM((NSRC,), jnp.int32),
        pltpu.VMEM((TOK_PER_W,), jnp.int32),
        pltpu.VMEM((NCHUNK, CHUNK), jnp.int32),
        pltpu.VMEM((NCHUNK, CHUNK), jnp.int32),
        pltpu.VMEM((TOK_PER_W + L,), jnp.float32),
        pltpu.VMEM((2, CHUNK, DIM), jnp.float32),
        pltpu.VMEM((2, CHUNK, DIM), jnp.float32),
        pltpu.SemaphoreType.DMA,
        pltpu.SemaphoreType.DMA,
    ],
)
def _sc_scatter(sidx_hbm, peidx_hbm, x_hbm, pe_hbm, out_hbm,
                idx_v, winner_v, wc_v, pw_v, valid_v, xrows_v, perows_v,
                semx, semp):
    wid = lax.axis_index("s") * NC + lax.axis_index("c")
    base = wid * TOK_PER_W

    pltpu.sync_copy(sidx_hbm, idx_v)
    neg1 = jnp.full((L,), -1, jnp.int32)

    def init_body(g, _):
        winner_v[pl.ds(g * L, L)] = neg1
        return 0

    lax.fori_loop(0, TOK_PER_W // L, init_body, 0)

    sent = jnp.int32(2**31 - 1)
    iota = lax.iota(jnp.int32, L)
    shift_idx = jnp.minimum(iota + 1, L - 1)
    last_lane = iota == (L - 1)

    def scan_body(g, _):
        idx16 = idx_v[pl.ds(g * L, L)]
        local = idx16 - base
        inr = (local >= 0) & (local < TOK_PER_W)
        i_vec = g * L + iota
        key = jnp.where(inr, (local << IDX_BITS) | i_vec, sent)
        skey, _ = plsc.sort_key_val(key, key)
        nxt = skey.at[shift_idx].get(mode="promise_in_bounds")
        tok = skey >> IDX_BITS
        keep = ((tok != (nxt >> IDX_BITS)) | last_lane) & (skey != sent)
        tok_st = tok & (TOK_PER_W - 1)
        ival = skey & (NSRC - 1)
        plsc.store_scatter(winner_v, [tok_st], ival, mask=keep)
        return 0

    lax.fori_loop(0, NSRC // L, scan_body, 0)

    pltpu.sync_copy(peidx_hbm, idx_v)

    def pw_body(g, nc_acc):
        sl = pl.ds(g * L, L)
        w16 = winner_v[sl]
        valid = w16 >= 0
        wcl = jnp.maximum(w16, 0)
        c = g // (CHUNK // L)
        o = (g % (CHUNK // L)) * L
        csl = pl.ds(o, L)
        wc_v[c, csl] = wcl
        pw = plsc.load_gather(idx_v, [wcl])
        pw_v[c, csl] = pw
        valid_v[sl] = jnp.where(valid, 1.0, 0.0).astype(jnp.float32)
        return nc_acc

    lax.fori_loop(0, TOK_PER_W // L, pw_body, 0)

    def issue(c):
        p = c % 2
        gx = pltpu.async_copy(x_hbm.at[wc_v.at[c]], xrows_v.at[p], semx)
        gp = pltpu.async_copy(pe_hbm.at[pw_v.at[c]], perows_v.at[p], semp)
        return gx, gp

    pend = issue(0)
    for c in range(NCHUNK):
        p = c % 2
        pend[0].wait()
        pend[1].wait()
        if c + 1 < NCHUNK:
            pend = issue(c + 1)

        def row_body(r, _, c=c, p=p):
            fvec = valid_v[pl.ds(c * CHUNK + r, L)]
            fv = jnp.broadcast_to(fvec[0], (L,))
            for j in range(DIM // L):
                sl = pl.ds(j * L, L)
                xrows_v[p, r, sl] = (
                    xrows_v[p, r, sl] + perows_v[p, r, sl]) * fv
            return 0

        lax.fori_loop(0, CHUNK, row_body, 0)
        pltpu.sync_copy(
            xrows_v.at[p], out_hbm.at[pl.ds(base + c * CHUNK, CHUNK)])


def kernel(sdata, scatter_idxs, pe_idxs, pe_embed, W, b):
    x = _matmul(sdata.reshape(-1, IN_FEAT), W, b)
    return _sc_scatter(
        scatter_idxs.astype(jnp.int32), pe_idxs.astype(jnp.int32), x, pe_embed)


# speedup vs baseline: 2.0905x; 1.8223x over previous
import functools

import jax
import jax.numpy as jnp
from jax import lax
from jax.experimental import pallas as pl
from jax.experimental.pallas import tpu as pltpu
from jax.experimental.pallas import tpu_sc as plsc

NUM_TOKENS = 32768
NSRC = 32768
IN_FEAT = 128
DIM = 256
NC, NS, L = 2, 16, 16
NW = NC * NS
TOK_PER_W = NUM_TOKENS // NW
IDX_BITS = 15
BT = 512


_mesh = plsc.VectorSubcoreMesh(core_axis_name="c", subcore_axis_name="s")


@functools.partial(
    pl.kernel,
    out_type=(
        jax.ShapeDtypeStruct((NUM_TOKENS,), jnp.int32),
        jax.ShapeDtypeStruct((NUM_TOKENS,), jnp.int32),
    ),
    mesh=_mesh,
    compiler_params=pltpu.CompilerParams(needs_layout_passes=False),
    scratch_types=[
        pltpu.VMEM((NSRC,), jnp.int32),
        pltpu.VMEM((TOK_PER_W,), jnp.int32),
        pltpu.VMEM((TOK_PER_W,), jnp.int32),
    ],
)
def _sc_winner(sidx_hbm, peidx_hbm, w_hbm, pw_hbm, idx_v, winner_v, pw_v):
    wid = lax.axis_index("s") * NC + lax.axis_index("c")
    base = wid * TOK_PER_W

    pltpu.sync_copy(sidx_hbm, idx_v)
    neg1 = jnp.full((L,), -1, jnp.int32)

    def init_body(g, _):
        winner_v[pl.ds(g * L, L)] = neg1
        return 0

    lax.fori_loop(0, TOK_PER_W // L, init_body, 0)

    sent = jnp.int32(2**31 - 1)
    iota = lax.iota(jnp.int32, L)
    shift_idx = jnp.minimum(iota + 1, L - 1)
    last_lane = iota == (L - 1)

    def scan_body(g, _):
        idx16 = idx_v[pl.ds(g * L, L)]
        local = idx16 - base
        inr = (local >= 0) & (local < TOK_PER_W)
        i_vec = g * L + iota
        key = jnp.where(inr, (local << IDX_BITS) | i_vec, sent)
        skey, _ = plsc.sort_key_val(key, key)
        nxt = skey.at[shift_idx].get(mode="promise_in_bounds")
        tok = skey >> IDX_BITS
        keep = ((tok != (nxt >> IDX_BITS)) | last_lane) & (skey != sent)
        tok_st = tok & (TOK_PER_W - 1)
        ival = skey & (NSRC - 1)
        plsc.store_scatter(winner_v, [tok_st], ival, mask=keep)
        return 0

    lax.fori_loop(0, NSRC // L, scan_body, 0)

    pltpu.sync_copy(peidx_hbm, idx_v)

    def pw_body(g, _):
        sl = pl.ds(g * L, L)
        w16 = winner_v[sl]
        wcl = jnp.maximum(w16, 0)
        pw_v[sl] = plsc.load_gather(idx_v, [wcl])
        return 0

    lax.fori_loop(0, TOK_PER_W // L, pw_body, 0)

    pltpu.sync_copy(winner_v, w_hbm.at[pl.ds(base, TOK_PER_W)])
    pltpu.sync_copy(pw_v, pw_hbm.at[pl.ds(base, TOK_PER_W)])


def _combine_body(w_ref, pw_ref, sdata_ref, wm_ref, b_ref, pe_ref,
                  o_ref, xg_ref):
    def gather_x(r, _):
        wr = w_ref[0, 0, r]
        idx = jnp.maximum(wr, 0)
        xg_ref[pl.ds(r, 1), :] = sdata_ref[pl.ds(idx, 1), :]
        return 0

    lax.fori_loop(0, BT, gather_x, 0, unroll=8)

    o_ref[...] = (
        jnp.dot(xg_ref[...], wm_ref[...], preferred_element_type=jnp.float32)
        + b_ref[...]
    )

    def add_pe(r, _):
        wr = w_ref[0, 0, r]
        pwr = pw_ref[0, 0, r]
        fv = jnp.where(wr >= 0, 1.0, 0.0).astype(jnp.float32)
        o_ref[pl.ds(r, 1), :] = (
            o_ref[pl.ds(r, 1), :] + pe_ref[pl.ds(pwr, 1), :]) * fv
        return 0

    lax.fori_loop(0, BT, add_pe, 0, unroll=8)


def _tc_combine(w_raw, pw, sdata2d, W, b, pe):
    nblk = NUM_TOKENS // BT
    return pl.pallas_call(
        _combine_body,
        grid=(nblk,),
        in_specs=[
            pl.BlockSpec((1, 1, BT), lambda i: (i, 0, 0),
                         memory_space=pltpu.SMEM),
            pl.BlockSpec((1, 1, BT), lambda i: (i, 0, 0),
                         memory_space=pltpu.SMEM),
            pl.BlockSpec((NSRC, IN_FEAT), lambda i: (0, 0)),
            pl.BlockSpec((IN_FEAT, DIM), lambda i: (0, 0)),
            pl.BlockSpec((1, DIM), lambda i: (0, 0)),
            pl.BlockSpec((NUM_TOKENS, DIM), lambda i: (0, 0)),
        ],
        out_specs=pl.BlockSpec((BT, DIM), lambda i: (i, 0)),
        out_shape=jax.ShapeDtypeStruct((NUM_TOKENS, DIM), jnp.float32),
        scratch_shapes=[pltpu.VMEM((BT, IN_FEAT), jnp.float32)],
        compiler_params=pltpu.CompilerParams(
            vmem_limit_bytes=56 * 1024 * 1024),
    )(w_raw.reshape(nblk, 1, BT), pw.reshape(nblk, 1, BT),
      sdata2d, W, b.reshape(1, DIM), pe)


def kernel(sdata, scatter_idxs, pe_idxs, pe_embed, W, b):
    w_raw, pw = _sc_winner(
        scatter_idxs.astype(jnp.int32), pe_idxs.astype(jnp.int32))
    return _tc_combine(
        w_raw, pw, sdata.reshape(-1, IN_FEAT), W, b, pe_embed)


# speedup vs baseline: 2.5027x; 1.1972x over previous
import functools

import jax
import jax.numpy as jnp
from jax import lax
from jax.experimental import pallas as pl
from jax.experimental.pallas import tpu as pltpu
from jax.experimental.pallas import tpu_sc as plsc

NUM_TOKENS = 32768
NSRC = 32768
IN_FEAT = 128
DIM = 256
NC, NS, L = 2, 16, 16
NW = NC * NS
TOK_PER_W = NUM_TOKENS // NW
IDX_BITS = 15
BT = 512


_mesh = plsc.VectorSubcoreMesh(core_axis_name="c", subcore_axis_name="s")


@functools.partial(
    pl.kernel,
    out_type=(
        jax.ShapeDtypeStruct((NUM_TOKENS,), jnp.int32),
        jax.ShapeDtypeStruct((NUM_TOKENS,), jnp.int32),
        jax.ShapeDtypeStruct((NUM_TOKENS,), jnp.float32),
    ),
    mesh=_mesh,
    compiler_params=pltpu.CompilerParams(needs_layout_passes=False),
    scratch_types=[
        pltpu.VMEM((NSRC,), jnp.int32),
        pltpu.VMEM((TOK_PER_W,), jnp.int32),
        pltpu.VMEM((TOK_PER_W,), jnp.int32),
        pltpu.VMEM((TOK_PER_W,), jnp.float32),
    ],
)
def _sc_winner(sidx_hbm, peidx_hbm, w_hbm, pw_hbm, v_hbm,
               idx_v, winner_v, pw_v, valid_v):
    wid = lax.axis_index("s") * NC + lax.axis_index("c")
    base = wid * TOK_PER_W

    pltpu.sync_copy(sidx_hbm, idx_v)
    neg1 = jnp.full((L,), -1, jnp.int32)

    def init_body(g, _):
        winner_v[pl.ds(g * L, L)] = neg1
        return 0

    lax.fori_loop(0, TOK_PER_W // L, init_body, 0)

    sent = jnp.int32(2**31 - 1)
    iota = lax.iota(jnp.int32, L)
    shift_idx = jnp.minimum(iota + 1, L - 1)
    last_lane = iota == (L - 1)

    def scan_body(g, _):
        idx16 = idx_v[pl.ds(g * L, L)]
        local = idx16 - base
        inr = (local >= 0) & (local < TOK_PER_W)
        i_vec = g * L + iota
        key = jnp.where(inr, (local << IDX_BITS) | i_vec, sent)
        skey, _ = plsc.sort_key_val(key, key)
        nxt = skey.at[shift_idx].get(mode="promise_in_bounds")
        tok = skey >> IDX_BITS
        keep = ((tok != (nxt >> IDX_BITS)) | last_lane) & (skey != sent)
        tok_st = tok & (TOK_PER_W - 1)
        ival = skey & (NSRC - 1)
        plsc.store_scatter(winner_v, [tok_st], ival, mask=keep)
        return 0

    lax.fori_loop(0, NSRC // L, scan_body, 0)

    pltpu.sync_copy(peidx_hbm, idx_v)

    def pw_body(g, _):
        sl = pl.ds(g * L, L)
        w16 = winner_v[sl]
        wcl = jnp.maximum(w16, 0)
        pw_v[sl] = plsc.load_gather(idx_v, [wcl])
        valid_v[sl] = jnp.where(w16 >= 0, 1.0, 0.0).astype(jnp.float32)
        return 0

    lax.fori_loop(0, TOK_PER_W // L, pw_body, 0)

    pltpu.sync_copy(winner_v, w_hbm.at[pl.ds(base, TOK_PER_W)])
    pltpu.sync_copy(pw_v, pw_hbm.at[pl.ds(base, TOK_PER_W)])
    pltpu.sync_copy(valid_v, v_hbm.at[pl.ds(base, TOK_PER_W)])


def _combine_body(w_ref, pw_ref, sdata_ref, wm_ref, b_ref, pe_ref, v_ref,
                  o_ref, xg_ref, peg_ref):
    def gather_x(r, _):
        wr = w_ref[0, 0, r]
        idx = jnp.maximum(wr, 0)
        xg_ref[pl.ds(r, 1), :] = sdata_ref[pl.ds(idx, 1), :]
        return 0

    lax.fori_loop(0, BT, gather_x, 0, unroll=8)

    def gather_pe(r, _):
        pwr = pw_ref[0, 0, r]
        peg_ref[pl.ds(r, 1), :] = pe_ref[pl.ds(pwr, 1), :]
        return 0

    lax.fori_loop(0, BT, gather_pe, 0, unroll=8)

    mm = (
        jnp.dot(xg_ref[...], wm_ref[...], preferred_element_type=jnp.float32)
        + b_ref[...]
    )
    o_ref[...] = (mm + peg_ref[...]) * v_ref[...]


def _tc_combine(w_raw, pw, valid, sdata2d, W, b, pe):
    nblk = NUM_TOKENS // BT
    return pl.pallas_call(
        _combine_body,
        grid=(nblk,),
        in_specs=[
            pl.BlockSpec((1, 1, BT), lambda i: (i, 0, 0),
                         memory_space=pltpu.SMEM),
            pl.BlockSpec((1, 1, BT), lambda i: (i, 0, 0),
                         memory_space=pltpu.SMEM),
            pl.BlockSpec((NSRC, IN_FEAT), lambda i: (0, 0)),
            pl.BlockSpec((IN_FEAT, DIM), lambda i: (0, 0)),
            pl.BlockSpec((1, DIM), lambda i: (0, 0)),
            pl.BlockSpec((NUM_TOKENS, DIM), lambda i: (0, 0)),
            pl.BlockSpec((BT, 1), lambda i: (i, 0)),
        ],
        out_specs=pl.BlockSpec((BT, DIM), lambda i: (i, 0)),
        out_shape=jax.ShapeDtypeStruct((NUM_TOKENS, DIM), jnp.float32),
        scratch_shapes=[
            pltpu.VMEM((BT, IN_FEAT), jnp.float32),
            pltpu.VMEM((BT, DIM), jnp.float32),
        ],
        compiler_params=pltpu.CompilerParams(
            vmem_limit_bytes=56 * 1024 * 1024),
    )(w_raw.reshape(nblk, 1, BT), pw.reshape(nblk, 1, BT),
      sdata2d, W, b.reshape(1, DIM), pe, valid.reshape(NUM_TOKENS, 1))


def kernel(sdata, scatter_idxs, pe_idxs, pe_embed, W, b):
    w_raw, pw, valid = _sc_winner(
        scatter_idxs.astype(jnp.int32), pe_idxs.astype(jnp.int32))
    return _tc_combine(
        w_raw, pw, valid, sdata.reshape(-1, IN_FEAT), W, b, pe_embed)


# speedup vs baseline: 2.6890x; 1.0744x over previous
import functools

import jax
import jax.numpy as jnp
from jax import lax
from jax.experimental import pallas as pl
from jax.experimental.pallas import tpu as pltpu
from jax.experimental.pallas import tpu_sc as plsc

NUM_TOKENS = 32768
NSRC = 32768
IN_FEAT = 128
DIM = 256
NC, NS, L = 2, 16, 16
NW = NC * NS
TOK_PER_W = NUM_TOKENS // NW
IDX_BITS = 15
BT = 1024


_mesh = plsc.VectorSubcoreMesh(core_axis_name="c", subcore_axis_name="s")


@functools.partial(
    pl.kernel,
    out_type=(
        jax.ShapeDtypeStruct((NUM_TOKENS,), jnp.int32),
        jax.ShapeDtypeStruct((NUM_TOKENS,), jnp.int32),
        jax.ShapeDtypeStruct((NUM_TOKENS,), jnp.float32),
    ),
    mesh=_mesh,
    compiler_params=pltpu.CompilerParams(needs_layout_passes=False),
    scratch_types=[
        pltpu.VMEM((NSRC,), jnp.int32),
        pltpu.VMEM((TOK_PER_W,), jnp.int32),
        pltpu.VMEM((TOK_PER_W,), jnp.int32),
        pltpu.VMEM((TOK_PER_W,), jnp.float32),
    ],
)
def _sc_winner(sidx_hbm, peidx_hbm, w_hbm, pw_hbm, v_hbm,
               idx_v, winner_v, pw_v, valid_v):
    wid = lax.axis_index("s") * NC + lax.axis_index("c")
    base = wid * TOK_PER_W

    pltpu.sync_copy(sidx_hbm, idx_v)
    neg1 = jnp.full((L,), -1, jnp.int32)

    def init_body(g, _):
        winner_v[pl.ds(g * L, L)] = neg1
        return 0

    lax.fori_loop(0, TOK_PER_W // L, init_body, 0)

    sent = jnp.int32(2**31 - 1)
    iota = lax.iota(jnp.int32, L)
    shift_idx = jnp.minimum(iota + 1, L - 1)
    last_lane = iota == (L - 1)

    def scan_body(g, _):
        idx16 = idx_v[pl.ds(g * L, L)]
        local = idx16 - base
        inr = (local >= 0) & (local < TOK_PER_W)
        i_vec = g * L + iota
        key = jnp.where(inr, (local << IDX_BITS) | i_vec, sent)
        skey, _ = plsc.sort_key_val(key, key)
        nxt = skey.at[shift_idx].get(mode="promise_in_bounds")
        tok = skey >> IDX_BITS
        keep = ((tok != (nxt >> IDX_BITS)) | last_lane) & (skey != sent)
        tok_st = tok & (TOK_PER_W - 1)
        ival = skey & (NSRC - 1)
        plsc.store_scatter(winner_v, [tok_st], ival, mask=keep)
        return 0

    lax.fori_loop(0, NSRC // L, scan_body, 0)

    pltpu.sync_copy(peidx_hbm, idx_v)

    def pw_body(g, _):
        sl = pl.ds(g * L, L)
        w16 = winner_v[sl]
        wcl = jnp.maximum(w16, 0)
        pw_v[sl] = plsc.load_gather(idx_v, [wcl])
        valid_v[sl] = jnp.where(w16 >= 0, 1.0, 0.0).astype(jnp.float32)
        return 0

    lax.fori_loop(0, TOK_PER_W // L, pw_body, 0)

    pltpu.sync_copy(winner_v, w_hbm.at[pl.ds(base, TOK_PER_W)])
    pltpu.sync_copy(pw_v, pw_hbm.at[pl.ds(base, TOK_PER_W)])
    pltpu.sync_copy(valid_v, v_hbm.at[pl.ds(base, TOK_PER_W)])


def _combine_body(w_ref, pw_ref, sdata_ref, wm_ref, b_ref, pe_ref, v_ref,
                  o_ref, xg_ref, peg_ref):
    def gather_x(r, _):
        wr = w_ref[0, 0, r]
        idx = jnp.maximum(wr, 0)
        xg_ref[pl.ds(r, 1), :] = sdata_ref[pl.ds(idx, 1), :]
        return 0

    lax.fori_loop(0, BT, gather_x, 0, unroll=16)

    def gather_pe(r, _):
        pwr = pw_ref[0, 0, r]
        peg_ref[pl.ds(r, 1), :] = pe_ref[pl.ds(pwr, 1), :]
        return 0

    lax.fori_loop(0, BT, gather_pe, 0, unroll=16)

    mm = (
        jnp.dot(xg_ref[...], wm_ref[...], preferred_element_type=jnp.float32)
        + b_ref[...]
    )
    o_ref[...] = (mm + peg_ref[...]) * v_ref[...]


def _tc_combine(w_raw, pw, valid, sdata2d, W, b, pe):
    nblk = NUM_TOKENS // BT
    return pl.pallas_call(
        _combine_body,
        grid=(nblk,),
        in_specs=[
            pl.BlockSpec((1, 1, BT), lambda i: (i, 0, 0),
                         memory_space=pltpu.SMEM),
            pl.BlockSpec((1, 1, BT), lambda i: (i, 0, 0),
                         memory_space=pltpu.SMEM),
            pl.BlockSpec((NSRC, IN_FEAT), lambda i: (0, 0)),
            pl.BlockSpec((IN_FEAT, DIM), lambda i: (0, 0)),
            pl.BlockSpec((1, DIM), lambda i: (0, 0)),
            pl.BlockSpec((NUM_TOKENS, DIM), lambda i: (0, 0)),
            pl.BlockSpec((BT, 1), lambda i: (i, 0)),
        ],
        out_specs=pl.BlockSpec((BT, DIM), lambda i: (i, 0)),
        out_shape=jax.ShapeDtypeStruct((NUM_TOKENS, DIM), jnp.float32),
        scratch_shapes=[
            pltpu.VMEM((BT, IN_FEAT), jnp.float32),
            pltpu.VMEM((BT, DIM), jnp.float32),
        ],
        compiler_params=pltpu.CompilerParams(
            vmem_limit_bytes=56 * 1024 * 1024),
    )(w_raw.reshape(nblk, 1, BT), pw.reshape(nblk, 1, BT),
      sdata2d, W, b.reshape(1, DIM), pe, valid.reshape(NUM_TOKENS, 1))


def kernel(sdata, scatter_idxs, pe_idxs, pe_embed, W, b):
    w_raw, pw, valid = _sc_winner(
        scatter_idxs.astype(jnp.int32), pe_idxs.astype(jnp.int32))
    return _tc_combine(
        w_raw, pw, valid, sdata.reshape(-1, IN_FEAT), W, b, pe_embed)


# speedup vs baseline: 2.6890x; 1.0000x over previous
import functools

import jax
import jax.numpy as jnp
from jax import lax
from jax.experimental import pallas as pl
from jax.experimental.pallas import tpu as pltpu
from jax.experimental.pallas import tpu_sc as plsc

NUM_TOKENS = 32768
NSRC = 32768
IN_FEAT = 128
DIM = 256
NC, NS, L = 2, 16, 16
NW = NC * NS
TOK_PER_W = NUM_TOKENS // NW
IDX_BITS = 15
BT = 1024


_mesh = plsc.VectorSubcoreMesh(core_axis_name="c", subcore_axis_name="s")


@functools.partial(
    pl.kernel,
    out_type=(
        jax.ShapeDtypeStruct((NUM_TOKENS,), jnp.int32),
        jax.ShapeDtypeStruct((NUM_TOKENS,), jnp.int32),
        jax.ShapeDtypeStruct((NUM_TOKENS,), jnp.float32),
    ),
    mesh=_mesh,
    compiler_params=pltpu.CompilerParams(needs_layout_passes=False),
    scratch_types=[
        pltpu.VMEM((NSRC,), jnp.int32),
        pltpu.VMEM((TOK_PER_W,), jnp.int32),
        pltpu.VMEM((TOK_PER_W,), jnp.int32),
        pltpu.VMEM((TOK_PER_W,), jnp.float32),
    ],
)
def _sc_winner(sidx_hbm, peidx_hbm, w_hbm, pw_hbm, v_hbm,
               idx_v, winner_v, pw_v, valid_v):
    wid = lax.axis_index("s") * NC + lax.axis_index("c")
    base = wid * TOK_PER_W

    pltpu.sync_copy(sidx_hbm, idx_v)
    neg1 = jnp.full((L,), -1, jnp.int32)

    def init_body(g, _):
        winner_v[pl.ds(g * L, L)] = neg1
        return 0

    lax.fori_loop(0, TOK_PER_W // L, init_body, 0)

    sent = jnp.int32(2**31 - 1)
    iota = lax.iota(jnp.int32, L)
    shift_idx = jnp.minimum(iota + 1, L - 1)
    last_lane = iota == (L - 1)

    def scan_body(g, _):
        idx16 = idx_v[pl.ds(g * L, L)]
        local = idx16 - base
        inr = (local >= 0) & (local < TOK_PER_W)
        i_vec = g * L + iota
        key = jnp.where(inr, (local << IDX_BITS) | i_vec, sent)
        skey, _ = plsc.sort_key_val(key, key)
        nxt = skey.at[shift_idx].get(mode="promise_in_bounds")
        tok = skey >> IDX_BITS
        keep = ((tok != (nxt >> IDX_BITS)) | last_lane) & (skey != sent)
        tok_st = tok & (TOK_PER_W - 1)
        ival = skey & (NSRC - 1)
        plsc.store_scatter(winner_v, [tok_st], ival, mask=keep)
        return 0

    lax.fori_loop(0, NSRC // L, scan_body, 0, unroll=4)

    pltpu.sync_copy(peidx_hbm, idx_v)

    def pw_body(g, _):
        sl = pl.ds(g * L, L)
        w16 = winner_v[sl]
        wcl = jnp.maximum(w16, 0)
        pw_v[sl] = plsc.load_gather(idx_v, [wcl])
        valid_v[sl] = jnp.where(w16 >= 0, 1.0, 0.0).astype(jnp.float32)
        return 0

    lax.fori_loop(0, TOK_PER_W // L, pw_body, 0)

    pltpu.sync_copy(winner_v, w_hbm.at[pl.ds(base, TOK_PER_W)])
    pltpu.sync_copy(pw_v, pw_hbm.at[pl.ds(base, TOK_PER_W)])
    pltpu.sync_copy(valid_v, v_hbm.at[pl.ds(base, TOK_PER_W)])


def _combine_body(w_ref, pw_ref, sdata_ref, wm_ref, b_ref, pe_ref, v_ref,
                  o_ref, xg_ref, peg_ref):
    def gather_x(r, _):
        wr = w_ref[0, 0, r]
        idx = jnp.maximum(wr, 0)
        xg_ref[pl.ds(r, 1), :] = sdata_ref[pl.ds(idx, 1), :]
        return 0

    lax.fori_loop(0, BT, gather_x, 0, unroll=16)

    def gather_pe(r, _):
        pwr = pw_ref[0, 0, r]
        peg_ref[pl.ds(r, 1), :] = pe_ref[pl.ds(pwr, 1), :]
        return 0

    lax.fori_loop(0, BT, gather_pe, 0, unroll=16)

    mm = (
        jnp.dot(xg_ref[...], wm_ref[...], preferred_element_type=jnp.float32)
        + b_ref[...]
    )
    o_ref[...] = (mm + peg_ref[...]) * v_ref[...]


def _tc_combine(w_raw, pw, valid, sdata2d, W, b, pe):
    nblk = NUM_TOKENS // BT
    return pl.pallas_call(
        _combine_body,
        grid=(nblk,),
        in_specs=[
            pl.BlockSpec((1, 1, BT), lambda i: (i, 0, 0),
                         memory_space=pltpu.SMEM),
            pl.BlockSpec((1, 1, BT), lambda i: (i, 0, 0),
                         memory_space=pltpu.SMEM),
            pl.BlockSpec((NSRC, IN_FEAT), lambda i: (0, 0)),
            pl.BlockSpec((IN_FEAT, DIM), lambda i: (0, 0)),
            pl.BlockSpec((1, DIM), lambda i: (0, 0)),
            pl.BlockSpec((NUM_TOKENS, DIM), lambda i: (0, 0)),
            pl.BlockSpec((BT, 1), lambda i: (i, 0)),
        ],
        out_specs=pl.BlockSpec((BT, DIM), lambda i: (i, 0)),
        out_shape=jax.ShapeDtypeStruct((NUM_TOKENS, DIM), jnp.float32),
        scratch_shapes=[
            pltpu.VMEM((BT, IN_FEAT), jnp.float32),
            pltpu.VMEM((BT, DIM), jnp.float32),
        ],
        compiler_params=pltpu.CompilerParams(
            vmem_limit_bytes=56 * 1024 * 1024),
    )(w_raw.reshape(nblk, 1, BT), pw.reshape(nblk, 1, BT),
      sdata2d, W, b.reshape(1, DIM), pe, valid.reshape(NUM_TOKENS, 1))


def kernel(sdata, scatter_idxs, pe_idxs, pe_embed, W, b):
    w_raw, pw, valid = _sc_winner(
        scatter_idxs.astype(jnp.int32), pe_idxs.astype(jnp.int32))
    return _tc_combine(
        w_raw, pw, valid, sdata.reshape(-1, IN_FEAT), W, b, pe_embed)
